# baseline (speedup 1.0000x reference)
import functools

import jax
import jax.numpy as jnp
from jax import lax
from jax.experimental import pallas as pl
from jax.experimental.pallas import tpu as pltpu
from jax.experimental.pallas import tpu_sc as plsc

N_NODES = 10000
N_EDGES = 160000
D = 256
DH = D // 2
NC = 2
NS = 16
NW = NC * NS
CHUNK = 192
NCH = 54
EPT = NCH * CHUNK
E_PAD = NS * EPT
ACC_ROWS = N_NODES + 16
ROWS_MAIN = 624
ROWS_TAIL = N_NODES - ROWS_MAIN * NS


def _sc_aggregate_body(x_hbm, src_hbm, dst_hbm, out_hbm,
                       acc, src_a, src_b, dst_a, dst_b, rows_a, rows_b,
                       gsem_a, gsem_b):
    c = lax.axis_index("c")
    s = lax.axis_index("s")

    base = s * ROWS_MAIN
    pltpu.sync_copy(x_hbm.at[c, pl.ds(base, ROWS_MAIN)],
                    acc.at[pl.ds(base, ROWS_MAIN)])

    @pl.when(s == NS - 1)
    def _init_tail():
        tb = NS * ROWS_MAIN
        pltpu.sync_copy(x_hbm.at[c, pl.ds(tb, ROWS_TAIL)],
                        acc.at[pl.ds(tb, ROWS_TAIL)])

    plsc.subcore_barrier()
    ebase = s * EPT

    def stage(j, src_ref, dst_ref):
        off = ebase + j * CHUNK
        pltpu.sync_copy(src_hbm.at[pl.ds(off, CHUNK)], src_ref)
        pltpu.sync_copy(dst_hbm.at[pl.ds(off, CHUNK)], dst_ref)

    def gather(src_ref, rows_ref, sem):
        return pltpu.async_copy(x_hbm.at[c].at[src_ref], rows_ref, sem)

    def step(jj, carry):
        j = 2 * jj
        stage(j, src_a, dst_a)
        ga = gather(src_a, rows_a, gsem_a)
        stage(j + 1, src_b, dst_b)
        gb = gather(src_b, rows_b, gsem_b)
        ga.wait()
        pltpu.sync_copy(rows_a, acc.at[dst_a], add=True)
        gb.wait()
        pltpu.sync_copy(rows_b, acc.at[dst_b], add=True)
        return carry

    lax.fori_loop(0, NCH // 2, step, 0)
    plsc.subcore_barrier()

    pltpu.sync_copy(acc.at[pl.ds(base, ROWS_MAIN)],
                    out_hbm.at[c, pl.ds(base, ROWS_MAIN)])

    @pl.when(s == NS - 1)
    def _out_tail():
        tb = NS * ROWS_MAIN
        pltpu.sync_copy(acc.at[pl.ds(tb, ROWS_TAIL)],
                        out_hbm.at[c, pl.ds(tb, ROWS_TAIL)])


@jax.jit
def _sc_aggregate(x_split, src_pad, dst_pad):
    mesh = plsc.VectorSubcoreMesh(core_axis_name="c", subcore_axis_name="s",
                                  num_cores=NC, num_subcores=NS)
    return pl.kernel(
        _sc_aggregate_body,
        out_type=jax.ShapeDtypeStruct((NC, N_NODES, DH), jnp.float32),
        mesh=mesh,
        scratch_types=[
            pltpu.VMEM_SHARED((ACC_ROWS, DH), jnp.float32),
            pltpu.VMEM((CHUNK,), jnp.int32),
            pltpu.VMEM((CHUNK,), jnp.int32),
            pltpu.VMEM((CHUNK,), jnp.int32),
            pltpu.VMEM((CHUNK,), jnp.int32),
            pltpu.VMEM((CHUNK, DH), jnp.float32),
            pltpu.VMEM((CHUNK, DH), jnp.float32),
            pltpu.SemaphoreType.DMA,
            pltpu.SemaphoreType.DMA,
        ],
    )(x_split, src_pad, dst_pad)


ROW_BLK = 1000


def _mlp_body_split_out(y_ref, wa_ref, ba_ref, wb_ref, bb_ref, out_ref):
    h = (jnp.dot(y_ref[0], wa_ref[0], preferred_element_type=jnp.float32)
         + jnp.dot(y_ref[1], wa_ref[1], preferred_element_type=jnp.float32)
         + ba_ref[...])
    h = jnp.maximum(h, 0.0)
    o = jnp.dot(h, wb_ref[...], preferred_element_type=jnp.float32) + bb_ref[...]
    o = jnp.maximum(o, 0.0)
    out_ref[0] = o[:, :DH]
    out_ref[1] = o[:, DH:]


def _mlp_body_flat_out(y_ref, wa_ref, ba_ref, wb_ref, bb_ref, out_ref):
    h = (jnp.dot(y_ref[0], wa_ref[0], preferred_element_type=jnp.float32)
         + jnp.dot(y_ref[1], wa_ref[1], preferred_element_type=jnp.float32)
         + ba_ref[...])
    h = jnp.maximum(h, 0.0)
    out_ref[...] = jnp.dot(h, wb_ref[...], preferred_element_type=jnp.float32) + bb_ref[...]


def _mlp(y_split, Wa, ba, Wb, bb, split_out):
    wa2 = Wa.reshape(NC, DH, D)
    grid = (N_NODES // ROW_BLK,)
    in_specs = [
        pl.BlockSpec((NC, ROW_BLK, DH), lambda i: (0, i, 0)),
        pl.BlockSpec((NC, DH, D), lambda i: (0, 0, 0)),
        pl.BlockSpec((D,), lambda i: (0,)),
        pl.BlockSpec((D, D), lambda i: (0, 0)),
        pl.BlockSpec((D,), lambda i: (0,)),
    ]
    if split_out:
        body = _mlp_body_split_out
        out_shape = jax.ShapeDtypeStruct((NC, N_NODES, DH), jnp.float32)
        out_spec = pl.BlockSpec((NC, ROW_BLK, DH), lambda i: (0, i, 0))
    else:
        body = _mlp_body_flat_out
        out_shape = jax.ShapeDtypeStruct((N_NODES, D), jnp.float32)
        out_spec = pl.BlockSpec((ROW_BLK, D), lambda i: (i, 0))
    return pl.pallas_call(
        body,
        grid=grid,
        in_specs=in_specs,
        out_specs=out_spec,
        out_shape=out_shape,
    )(y_split, wa2, ba, Wb, bb)


def kernel(x, edge_index, W1a, b1a, W1b, b1b, W2a, b2a, W2b, b2b):
    src = edge_index[0].astype(jnp.int32)
    dst = edge_index[1].astype(jnp.int32)
    pad = E_PAD - N_EDGES
    src_pad = jnp.concatenate([src, jnp.zeros((pad,), jnp.int32)])
    pad_dst = N_NODES + (jnp.arange(pad, dtype=jnp.int32) % (ACC_ROWS - N_NODES))
    dst_pad = jnp.concatenate([dst, pad_dst])
    x_split = x.reshape(N_NODES, NC, DH).transpose(1, 0, 2)

    y1 = _sc_aggregate(x_split, src_pad, dst_pad)
    h1 = _mlp(y1, W1a, b1a, W1b, b1b, split_out=True)
    y2 = _sc_aggregate(h1, src_pad, dst_pad)
    out = _mlp(y2, W2a, b2a, W2b, b2b, split_out=False)
    return out


# speedup vs baseline: 2.6742x; 2.6742x over previous
import functools

import jax
import jax.numpy as jnp
from jax import lax
from jax.experimental import pallas as pl
from jax.experimental.pallas import tpu as pltpu
from jax.experimental.pallas import tpu_sc as plsc

N_NODES = 10000
N_EDGES = 160000
D = 256
DH = D // 2
NC = 2
NS = 16
NW = NC * NS
CHUNK = 192
NCH = 54
EPT = NCH * CHUNK
E_PAD = NS * EPT
ACC_ROWS = N_NODES + 16
ROWS_MAIN = 624
ROWS_TAIL = N_NODES - ROWS_MAIN * NS


def _sc_aggregate_body(x_hbm, src_hbm, dst_hbm, out_hbm,
                       acc, src_a, src_b, dst_a, dst_b, rows_a, rows_b,
                       gsem_a, gsem_b):
    c = lax.axis_index("c")
    s = lax.axis_index("s")

    base = s * ROWS_MAIN
    pltpu.sync_copy(x_hbm.at[c, pl.ds(base, ROWS_MAIN)],
                    acc.at[pl.ds(base, ROWS_MAIN)])

    @pl.when(s == NS - 1)
    def _init_tail():
        tb = NS * ROWS_MAIN
        pltpu.sync_copy(x_hbm.at[c, pl.ds(tb, ROWS_TAIL)],
                        acc.at[pl.ds(tb, ROWS_TAIL)])

    plsc.subcore_barrier()
    ebase = s * EPT

    def stage(j, src_ref, dst_ref):
        off = ebase + j * CHUNK
        pltpu.sync_copy(src_hbm.at[pl.ds(off, CHUNK)], src_ref)
        pltpu.sync_copy(dst_hbm.at[pl.ds(off, CHUNK)], dst_ref)

    def gather(src_ref, rows_ref, sem):
        return pltpu.async_copy(x_hbm.at[c].at[src_ref], rows_ref, sem)

    def step(jj, carry):
        j = 2 * jj
        stage(j, src_a, dst_a)
        ga = gather(src_a, rows_a, gsem_a)
        stage(j + 1, src_b, dst_b)
        gb = gather(src_b, rows_b, gsem_b)
        ga.wait()
        pltpu.sync_copy(rows_a, acc.at[dst_a], add=True)
        gb.wait()
        pltpu.sync_copy(rows_b, acc.at[dst_b], add=True)
        return carry

    lax.fori_loop(0, NCH // 2, step, 0)
    plsc.subcore_barrier()

    pltpu.sync_copy(acc.at[pl.ds(base, ROWS_MAIN)],
                    out_hbm.at[c, pl.ds(base, ROWS_MAIN)])

    @pl.when(s == NS - 1)
    def _out_tail():
        tb = NS * ROWS_MAIN
        pltpu.sync_copy(acc.at[pl.ds(tb, ROWS_TAIL)],
                        out_hbm.at[c, pl.ds(tb, ROWS_TAIL)])


@jax.jit
def _sc_aggregate(x_split, src_pad, dst_pad):
    mesh = plsc.VectorSubcoreMesh(core_axis_name="c", subcore_axis_name="s",
                                  num_cores=NC, num_subcores=NS)
    return pl.kernel(
        _sc_aggregate_body,
        out_type=jax.ShapeDtypeStruct((NC, N_NODES, DH), jnp.float32),
        mesh=mesh,
        scratch_types=[
            pltpu.VMEM_SHARED((ACC_ROWS, DH), jnp.float32),
            pltpu.VMEM((CHUNK,), jnp.int32),
            pltpu.VMEM((CHUNK,), jnp.int32),
            pltpu.VMEM((CHUNK,), jnp.int32),
            pltpu.VMEM((CHUNK,), jnp.int32),
            pltpu.VMEM((CHUNK, DH), jnp.float32),
            pltpu.VMEM((CHUNK, DH), jnp.float32),
            pltpu.SemaphoreType.DMA,
            pltpu.SemaphoreType.DMA,
        ],
    )(x_split, src_pad, dst_pad)


ROW_BLK = 1000


def _mlp_body_split_out(y_ref, wa_ref, ba_ref, wb_ref, bb_ref, out_ref):
    h = (jnp.dot(y_ref[0], wa_ref[0], preferred_element_type=jnp.float32)
         + jnp.dot(y_ref[1], wa_ref[1], preferred_element_type=jnp.float32)
         + ba_ref[...])
    h = jnp.maximum(h, 0.0)
    o = jnp.dot(h, wb_ref[...], preferred_element_type=jnp.float32) + bb_ref[...]
    o = jnp.maximum(o, 0.0)
    out_ref[0] = o[:, :DH]
    out_ref[1] = o[:, DH:]


def _mlp_body_flat_out(y_ref, wa_ref, ba_ref, wb_ref, bb_ref, out_ref):
    h = (jnp.dot(y_ref[0], wa_ref[0], preferred_element_type=jnp.float32)
         + jnp.dot(y_ref[1], wa_ref[1], preferred_element_type=jnp.float32)
         + ba_ref[...])
    h = jnp.maximum(h, 0.0)
    out_ref[...] = jnp.dot(h, wb_ref[...], preferred_element_type=jnp.float32) + bb_ref[...]


def _mlp(y_split, Wa, ba, Wb, bb, split_out):
    wa2 = Wa.reshape(NC, DH, D)
    grid = (N_NODES // ROW_BLK,)
    in_specs = [
        pl.BlockSpec((NC, ROW_BLK, DH), lambda i: (0, i, 0)),
        pl.BlockSpec((NC, DH, D), lambda i: (0, 0, 0)),
        pl.BlockSpec((D,), lambda i: (0,)),
        pl.BlockSpec((D, D), lambda i: (0, 0)),
        pl.BlockSpec((D,), lambda i: (0,)),
    ]
    if split_out:
        body = _mlp_body_split_out
        out_shape = jax.ShapeDtypeStruct((NC, N_NODES, DH), jnp.float32)
        out_spec = pl.BlockSpec((NC, ROW_BLK, DH), lambda i: (0, i, 0))
    else:
        body = _mlp_body_flat_out
        out_shape = jax.ShapeDtypeStruct((N_NODES, D), jnp.float32)
        out_spec = pl.BlockSpec((ROW_BLK, D), lambda i: (i, 0))
    return pl.pallas_call(
        body,
        grid=grid,
        in_specs=in_specs,
        out_specs=out_spec,
        out_shape=out_shape,
    )(y_split, wa2, ba, Wb, bb)


def kernel(x, edge_index, W1a, b1a, W1b, b1b, W2a, b2a, W2b, b2b):
    src = edge_index[0].astype(jnp.int32)
    dst = edge_index[1].astype(jnp.int32)
    pad = E_PAD - N_EDGES
    pad_src = jnp.arange(pad, dtype=jnp.int32) % N_NODES
    src_pad = jnp.concatenate([src, pad_src])
    pad_dst = N_NODES + (jnp.arange(pad, dtype=jnp.int32) % (ACC_ROWS - N_NODES))
    dst_pad = jnp.concatenate([dst, pad_dst])
    x_split = x.reshape(N_NODES, NC, DH).transpose(1, 0, 2)

    y1 = _sc_aggregate(x_split, src_pad, dst_pad)
    h1 = _mlp(y1, W1a, b1a, W1b, b1b, split_out=True)
    y2 = _sc_aggregate(h1, src_pad, dst_pad)
    out = _mlp(y2, W2a, b2a, W2b, b2b, split_out=False)
    return out
